# Initial kernel scaffold; baseline (speedup 1.0000x reference)
import jax
import jax.numpy as jnp
from jax.experimental import pallas as pl


def kernel(x, edge_attr, params, edge_index):
    raise NotImplementedError("write your pallas kernel here")


# speedup vs baseline: 3.0933x; 3.0933x over previous
import functools

import jax
import jax.numpy as jnp
from jax import lax
from jax.experimental import pallas as pl
from jax.experimental.pallas import tpu as pltpu
from jax.experimental.pallas import tpu_sc as plsc

_N = 10000
_E = 320000
_LAT = 128
_OUT = 3

_NC = 2
_NS = 16
_NW = _NC * _NS
_EPW = _E // _NW
_GC = 80
_NP = 10240
_RPS = _NP // _NS
_ZR = 128

_RB_E = 2000
_RB_N = 2000

_f32 = jnp.float32


def _ln(t, g, b):
    m = jnp.mean(t, axis=-1, keepdims=True)
    v = jnp.mean((t - m) * (t - m), axis=-1, keepdims=True)
    return (t - m) * lax.rsqrt(v + 1e-5) * g + b


def _dot(a, w):
    return jnp.dot(a, w, preferred_element_type=_f32)


def _enc_node_body(x, w1, b1, w2, b2, w3, b3, w4, b4, lg, lb, wa, wb,
                   h_o, a_o, b_o):
    t = jnp.maximum(_dot(x[...], w1[...]) + b1[...], 0)
    t = jnp.maximum(_dot(t, w2[...]) + b2[...], 0)
    t = jnp.maximum(_dot(t, w3[...]) + b3[...], 0)
    t = _dot(t, w4[...]) + b4[...]
    h = _ln(t, lg[...], lb[...])
    h_o[...] = h
    a_o[...] = _dot(h, wa[...])
    b_o[...] = _dot(h, wb[...])


def _enc_node_tc(x, mlp, lnp, wa, wb):
    g = _N // _RB_N
    d = pl.BlockSpec((_RB_N, _LAT), lambda i: (i, 0))
    w = pl.BlockSpec((_LAT, _LAT), lambda i: (0, 0))
    w1 = pl.BlockSpec((_LAT, _LAT), lambda i: (0, 0))
    v = pl.BlockSpec((1, _LAT), lambda i: (0, 0))
    return pl.pallas_call(
        _enc_node_body,
        grid=(g,),
        in_specs=[d, w1, v, w, v, w, v, w, v, v, v, w, w],
        out_specs=[d, d, d],
        out_shape=[jax.ShapeDtypeStruct((_N, _LAT), _f32)] * 3,
    )(x, mlp[0]["W"], mlp[0]["b"], mlp[1]["W"], mlp[1]["b"],
      mlp[2]["W"], mlp[2]["b"], mlp[3]["W"], mlp[3]["b"],
      lnp["g"], lnp["b"], wa, wb)


def _enc_edge_body(ea, w1, b1, w2, b2, w3, b3, w4, b4, lg, lb, e_o):
    t = jnp.maximum(_dot(ea[...], w1[...]) + b1[...], 0)
    t = jnp.maximum(_dot(t, w2[...]) + b2[...], 0)
    t = jnp.maximum(_dot(t, w3[...]) + b3[...], 0)
    t = _dot(t, w4[...]) + b4[...]
    e_o[...] = _ln(t, lg[...], lb[...])


def _enc_edge_tc(ea, mlp, lnp):
    g = _E // _RB_E
    din = pl.BlockSpec((_RB_E, 16), lambda i: (i, 0))
    d = pl.BlockSpec((_RB_E, _LAT), lambda i: (i, 0))
    w16 = pl.BlockSpec((16, _LAT), lambda i: (0, 0))
    w = pl.BlockSpec((_LAT, _LAT), lambda i: (0, 0))
    v = pl.BlockSpec((1, _LAT), lambda i: (0, 0))
    return pl.pallas_call(
        _enc_edge_body,
        grid=(g,),
        in_specs=[din, w16, v, w, v, w, v, w, v, v, v],
        out_specs=d,
        out_shape=jax.ShapeDtypeStruct((_E, _LAT), _f32),
    )(ea, mlp[0]["W"], mlp[0]["b"], mlp[1]["W"], mlp[1]["b"],
      mlp[2]["W"], mlp[2]["b"], mlp[3]["W"], mlp[3]["b"],
      lnp["g"], lnp["b"])


def _edge_blk_body(g1, g2, e, wc, b1, w2, b2, w3, b3, w4, b4, lg, lb,
                   msg_o, enew_o):
    t = g1[...] + g2[...] + _dot(e[...], wc[...]) + b1[...]
    t = jnp.maximum(t, 0)
    t = jnp.maximum(_dot(t, w2[...]) + b2[...], 0)
    t = jnp.maximum(_dot(t, w3[...]) + b3[...], 0)
    t = _dot(t, w4[...]) + b4[...]
    m = _ln(t, lg[...], lb[...])
    msg_o[...] = m
    enew_o[...] = m + e[...]


def _edge_blk_tc(g1, g2, e, wc, mlp, lnp):
    g = _E // _RB_E
    d = pl.BlockSpec((_RB_E, _LAT), lambda i: (i, 0))
    w = pl.BlockSpec((_LAT, _LAT), lambda i: (0, 0))
    v = pl.BlockSpec((1, _LAT), lambda i: (0, 0))
    return pl.pallas_call(
        _edge_blk_body,
        grid=(g,),
        in_specs=[d, d, d, w, v, w, v, w, v, w, v, v, v],
        out_specs=[d, d],
        out_shape=[jax.ShapeDtypeStruct((_E, _LAT), _f32)] * 2,
    )(g1, g2, e, wc, mlp[0]["b"], mlp[1]["W"], mlp[1]["b"],
      mlp[2]["W"], mlp[2]["b"], mlp[3]["W"], mlp[3]["b"],
      lnp["g"], lnp["b"])


def _node_mid_body(h, a0, a1, wna, wnb, b1, w2, b2, w3, b3, w4, b4, lg, lb,
                   wa, wb, h_o, a_o, b_o):
    agg = a0[...] + a1[...]
    t = jnp.maximum(_dot(h[...], wna[...]) + _dot(agg, wnb[...]) + b1[...], 0)
    t = jnp.maximum(_dot(t, w2[...]) + b2[...], 0)
    t = jnp.maximum(_dot(t, w3[...]) + b3[...], 0)
    t = _dot(t, w4[...]) + b4[...]
    hn = _ln(t, lg[...], lb[...]) + h[...]
    h_o[...] = hn
    a_o[...] = _dot(hn, wa[...])
    b_o[...] = _dot(hn, wb[...])


def _node_mid_tc(h, a0, a1, wna, wnb, mlp, lnp, wa, wb):
    g = _N // _RB_N
    d = pl.BlockSpec((_RB_N, _LAT), lambda i: (i, 0))
    w = pl.BlockSpec((_LAT, _LAT), lambda i: (0, 0))
    v = pl.BlockSpec((1, _LAT), lambda i: (0, 0))
    return pl.pallas_call(
        _node_mid_body,
        grid=(g,),
        in_specs=[d, d, d, w, w, v, w, v, w, v, w, v, v, v, w, w],
        out_specs=[d, d, d],
        out_shape=[jax.ShapeDtypeStruct((_N, _LAT), _f32)] * 3,
    )(h, a0, a1, wna, wnb, mlp[0]["b"], mlp[1]["W"], mlp[1]["b"],
      mlp[2]["W"], mlp[2]["b"], mlp[3]["W"], mlp[3]["b"],
      lnp["g"], lnp["b"], wa, wb)


def _node_fin_body(h, a0, a1, wna, wnb, b1, w2, b2, w3, b3, w4, b4, lg, lb,
                   d1w, d1b, d2w, d2b, d3w, d3b, d4w, d4b, y_o):
    agg = a0[...] + a1[...]
    t = jnp.maximum(_dot(h[...], wna[...]) + _dot(agg, wnb[...]) + b1[...], 0)
    t = jnp.maximum(_dot(t, w2[...]) + b2[...], 0)
    t = jnp.maximum(_dot(t, w3[...]) + b3[...], 0)
    t = _dot(t, w4[...]) + b4[...]
    hn = _ln(t, lg[...], lb[...]) + h[...]
    t = jnp.maximum(_dot(hn, d1w[...]) + d1b[...], 0)
    t = jnp.maximum(_dot(t, d2w[...]) + d2b[...], 0)
    t = jnp.maximum(_dot(t, d3w[...]) + d3b[...], 0)
    y_o[...] = _dot(t, d4w[...]) + d4b[...]


def _node_fin_tc(h, a0, a1, wna, wnb, mlp, lnp, dec):
    g = _N // _RB_N
    d = pl.BlockSpec((_RB_N, _LAT), lambda i: (i, 0))
    w = pl.BlockSpec((_LAT, _LAT), lambda i: (0, 0))
    v = pl.BlockSpec((1, _LAT), lambda i: (0, 0))
    wo = pl.BlockSpec((_LAT, _OUT), lambda i: (0, 0))
    vo = pl.BlockSpec((1, _OUT), lambda i: (0, 0))
    do = pl.BlockSpec((_RB_N, _OUT), lambda i: (i, 0))
    return pl.pallas_call(
        _node_fin_body,
        grid=(g,),
        in_specs=[d, d, d, w, w, v, w, v, w, v, w, v, v, v,
                  w, v, w, v, w, v, wo, vo],
        out_specs=do,
        out_shape=jax.ShapeDtypeStruct((_N, _OUT), _f32),
    )(h, a0, a1, wna, wnb, mlp[0]["b"], mlp[1]["W"], mlp[1]["b"],
      mlp[2]["W"], mlp[2]["b"], mlp[3]["W"], mlp[3]["b"],
      lnp["g"], lnp["b"],
      dec[0]["W"], dec[0]["b"], dec[1]["W"], dec[1]["b"],
      dec[2]["W"], dec[2]["b"], dec[3]["W"], dec[3]["b"])


def _sc_gather2(a_tab, b_tab, dst, src):
    mesh = plsc.VectorSubcoreMesh(core_axis_name="c", subcore_axis_name="s")

    @functools.partial(
        pl.kernel, mesh=mesh,
        out_type=[jax.ShapeDtypeStruct((_E, _LAT), _f32),
                  jax.ShapeDtypeStruct((_E, _LAT), _f32)],
        scratch_types=[pltpu.VMEM((_GC,), jnp.int32),
                       pltpu.VMEM((_GC,), jnp.int32),
                       pltpu.VMEM((_GC, _LAT), _f32),
                       pltpu.VMEM((_GC, _LAT), _f32),
                       pltpu.SemaphoreType.DMA,
                       pltpu.SemaphoreType.DMA],
    )
    def k(a_hbm, b_hbm, dst_hbm, src_hbm, g1_hbm, g2_hbm,
          idx1, idx2, buf1, buf2, s1, s2):
        wid = lax.axis_index("s") * _NC + lax.axis_index("c")

        def body(i, carry):
            base = wid * _EPW + i * _GC
            pltpu.sync_copy(dst_hbm.at[pl.ds(base, _GC)], idx1)
            pltpu.sync_copy(src_hbm.at[pl.ds(base, _GC)], idx2)
            c1 = pltpu.async_copy(a_hbm.at[idx1], buf1, s1)
            c2 = pltpu.async_copy(b_hbm.at[idx2], buf2, s2)
            c1.wait()
            c2.wait()
            pltpu.sync_copy(buf1, g1_hbm.at[pl.ds(base, _GC)])
            pltpu.sync_copy(buf2, g2_hbm.at[pl.ds(base, _GC)])
            return carry

        lax.fori_loop(0, _EPW // _GC, body, 0)

    return k(a_tab, b_tab, dst, src)


def _sc_segsum(msg, dst):
    mesh = plsc.VectorSubcoreMesh(core_axis_name="c", subcore_axis_name="s")

    @functools.partial(
        pl.kernel, mesh=mesh,
        out_type=[jax.ShapeDtypeStruct((_NP, _LAT), _f32),
                  jax.ShapeDtypeStruct((_NP, _LAT), _f32)],
        scratch_types=[pltpu.VMEM((_GC,), jnp.int32),
                       pltpu.VMEM((_GC, _LAT), _f32),
                       pltpu.VMEM((_ZR, _LAT), _f32),
                       pltpu.VMEM_SHARED((_NP, _LAT), _f32)],
    )
    def k(msg_hbm, dst_hbm, out0_hbm, out1_hbm, idx, buf, zbuf, acc):
        cid = lax.axis_index("c")
        sid = lax.axis_index("s")

        def zb(i, carry):
            zbuf[i // 8, pl.ds((i % 8) * 16, 16)] = jnp.zeros((16,), _f32)
            return carry

        lax.fori_loop(0, _ZR * 8, zb, 0)

        def za(j, carry):
            pltpu.sync_copy(zbuf, acc.at[pl.ds(sid * _RPS + j * _ZR, _ZR)])
            return carry

        lax.fori_loop(0, _RPS // _ZR, za, 0)
        plsc.subcore_barrier()

        base0 = cid * (_E // _NC) + sid * _EPW

        def body(i, carry):
            base = base0 + i * _GC
            pltpu.sync_copy(dst_hbm.at[pl.ds(base, _GC)], idx)
            pltpu.sync_copy(msg_hbm.at[pl.ds(base, _GC)], buf)
            pltpu.sync_copy(buf, acc.at[idx], add=True)
            return carry

        lax.fori_loop(0, _EPW // _GC, body, 0)
        plsc.subcore_barrier()

        @pl.when(cid == 0)
        def _():
            pltpu.sync_copy(acc.at[pl.ds(sid * _RPS, _RPS)],
                            out0_hbm.at[pl.ds(sid * _RPS, _RPS)])

        @pl.when(cid == 1)
        def _():
            pltpu.sync_copy(acc.at[pl.ds(sid * _RPS, _RPS)],
                            out1_hbm.at[pl.ds(sid * _RPS, _RPS)])

    return k(msg, dst)


def kernel(x, edge_attr, params, edge_index):
    src = edge_index[0]
    dst = edge_index[1]
    blocks = params["blocks"]

    def row(v):
        return v.reshape(1, -1)

    def prep_mlp(mlp):
        return [{"W": p["W"], "b": row(p["b"])} for p in mlp]

    def prep_ln(p):
        return {"g": row(p["g"]), "b": row(p["b"])}

    enc_n = prep_mlp(params["enc_node_mlp"])
    enc_e = prep_mlp(params["enc_edge_mlp"])
    dec = prep_mlp(params["dec_mlp"])
    enc_n_ln = prep_ln(params["enc_node_ln"])
    enc_e_ln = prep_ln(params["enc_edge_ln"])

    w1 = [blocks[s]["edge_mlp"][0]["W"] for s in range(2)]
    wa = [w[:_LAT] for w in w1]
    wb = [w[_LAT:2 * _LAT] for w in w1]
    wc = [w[2 * _LAT:] for w in w1]
    wn = [blocks[s]["node_mlp"][0]["W"] for s in range(2)]
    wna = [w[:_LAT] for w in wn]
    wnb = [w[_LAT:] for w in wn]
    e_mlp = [prep_mlp(blocks[s]["edge_mlp"]) for s in range(2)]
    n_mlp = [prep_mlp(blocks[s]["node_mlp"]) for s in range(2)]
    e_ln = [prep_ln(blocks[s]["edge_ln"]) for s in range(2)]
    n_ln = [prep_ln(blocks[s]["node_ln"]) for s in range(2)]

    h, a_tab, b_tab = _enc_node_tc(x, enc_n, enc_n_ln, wa[0], wb[0])
    e = _enc_edge_tc(edge_attr, enc_e, enc_e_ln)

    g1, g2 = _sc_gather2(a_tab, b_tab, dst, src)
    msg, e = _edge_blk_tc(g1, g2, e, wc[0], e_mlp[0], e_ln[0])
    agg0, agg1 = _sc_segsum(msg, dst)
    h, a_tab, b_tab = _node_mid_tc(h, agg0, agg1, wna[0], wnb[0],
                                   n_mlp[0], n_ln[0], wa[1], wb[1])

    g1, g2 = _sc_gather2(a_tab, b_tab, dst, src)
    msg, e = _edge_blk_tc(g1, g2, e, wc[1], e_mlp[1], e_ln[1])
    agg0, agg1 = _sc_segsum(msg, dst)
    y = _node_fin_tc(h, agg0, agg1, wna[1], wnb[1],
                     n_mlp[1], n_ln[1], dec)
    return y


# speedup vs baseline: 3.6028x; 1.1647x over previous
import functools

import jax
import jax.numpy as jnp
from jax import lax
from jax.experimental import pallas as pl
from jax.experimental.pallas import tpu as pltpu
from jax.experimental.pallas import tpu_sc as plsc

_N = 10000
_E = 320000
_LAT = 128
_OUT = 3

_NC = 2
_NS = 16
_NW = _NC * _NS
_EPW = _E // _NW
_GC = 80
_NP = 10240
_RPS = _NP // _NS
_ZR = 128

_RB_E = 2000
_RB_N = 2000

_f32 = jnp.float32


def _ln(t, g, b):
    m = jnp.mean(t, axis=-1, keepdims=True)
    v = jnp.mean((t - m) * (t - m), axis=-1, keepdims=True)
    return (t - m) * lax.rsqrt(v + 1e-5) * g + b


def _dot(a, w):
    return jnp.dot(a, w, preferred_element_type=_f32)


def _enc_node_body(x, w1, b1, w2, b2, w3, b3, w4, b4, lg, lb, wa, wb,
                   h_o, a_o, b_o):
    t = jnp.maximum(_dot(x[...], w1[...]) + b1[...], 0)
    t = jnp.maximum(_dot(t, w2[...]) + b2[...], 0)
    t = jnp.maximum(_dot(t, w3[...]) + b3[...], 0)
    t = _dot(t, w4[...]) + b4[...]
    h = _ln(t, lg[...], lb[...])
    h_o[...] = h
    a_o[...] = _dot(h, wa[...])
    b_o[...] = _dot(h, wb[...])


def _enc_node_tc(x, mlp, lnp, wa, wb):
    g = _N // _RB_N
    d = pl.BlockSpec((_RB_N, _LAT), lambda i: (i, 0))
    w = pl.BlockSpec((_LAT, _LAT), lambda i: (0, 0))
    w1 = pl.BlockSpec((_LAT, _LAT), lambda i: (0, 0))
    v = pl.BlockSpec((1, _LAT), lambda i: (0, 0))
    return pl.pallas_call(
        _enc_node_body,
        grid=(g,),
        in_specs=[d, w1, v, w, v, w, v, w, v, v, v, w, w],
        out_specs=[d, d, d],
        out_shape=[jax.ShapeDtypeStruct((_N, _LAT), _f32)] * 3,
    )(x, mlp[0]["W"], mlp[0]["b"], mlp[1]["W"], mlp[1]["b"],
      mlp[2]["W"], mlp[2]["b"], mlp[3]["W"], mlp[3]["b"],
      lnp["g"], lnp["b"], wa, wb)


def _enc_edge_body(ea, w1, b1, w2, b2, w3, b3, w4, b4, lg, lb, e_o):
    t = jnp.maximum(_dot(ea[...], w1[...]) + b1[...], 0)
    t = jnp.maximum(_dot(t, w2[...]) + b2[...], 0)
    t = jnp.maximum(_dot(t, w3[...]) + b3[...], 0)
    t = _dot(t, w4[...]) + b4[...]
    e_o[...] = _ln(t, lg[...], lb[...])


def _enc_edge_tc(ea, mlp, lnp):
    g = _E // _RB_E
    din = pl.BlockSpec((_RB_E, 16), lambda i: (i, 0))
    d = pl.BlockSpec((_RB_E, _LAT), lambda i: (i, 0))
    w16 = pl.BlockSpec((16, _LAT), lambda i: (0, 0))
    w = pl.BlockSpec((_LAT, _LAT), lambda i: (0, 0))
    v = pl.BlockSpec((1, _LAT), lambda i: (0, 0))
    return pl.pallas_call(
        _enc_edge_body,
        grid=(g,),
        in_specs=[din, w16, v, w, v, w, v, w, v, v, v],
        out_specs=d,
        out_shape=jax.ShapeDtypeStruct((_E, _LAT), _f32),
    )(ea, mlp[0]["W"], mlp[0]["b"], mlp[1]["W"], mlp[1]["b"],
      mlp[2]["W"], mlp[2]["b"], mlp[3]["W"], mlp[3]["b"],
      lnp["g"], lnp["b"])


def _edge_blk_body(g1, e, wc, b1, w2, b2, w3, b3, w4, b4, lg, lb,
                   msg_o, enew_o):
    t = g1[...] + _dot(e[...], wc[...]) + b1[...]
    t = jnp.maximum(t, 0)
    t = jnp.maximum(_dot(t, w2[...]) + b2[...], 0)
    t = jnp.maximum(_dot(t, w3[...]) + b3[...], 0)
    t = _dot(t, w4[...]) + b4[...]
    m = _ln(t, lg[...], lb[...])
    msg_o[...] = m
    enew_o[...] = m + e[...]


def _edge_blk_tc(g1, e, wc, mlp, lnp):
    g = _E // _RB_E
    d = pl.BlockSpec((_RB_E, _LAT), lambda i: (i, 0))
    w = pl.BlockSpec((_LAT, _LAT), lambda i: (0, 0))
    v = pl.BlockSpec((1, _LAT), lambda i: (0, 0))
    return pl.pallas_call(
        _edge_blk_body,
        grid=(g,),
        in_specs=[d, d, w, v, w, v, w, v, w, v, v, v],
        out_specs=[d, d],
        out_shape=[jax.ShapeDtypeStruct((_E, _LAT), _f32)] * 2,
    )(g1, e, wc, mlp[0]["b"], mlp[1]["W"], mlp[1]["b"],
      mlp[2]["W"], mlp[2]["b"], mlp[3]["W"], mlp[3]["b"],
      lnp["g"], lnp["b"])


def _node_mid_body(h, a0, a1, wna, wnb, b1, w2, b2, w3, b3, w4, b4, lg, lb,
                   wa, wb, h_o, a_o, b_o):
    agg = a0[...] + a1[...]
    t = jnp.maximum(_dot(h[...], wna[...]) + _dot(agg, wnb[...]) + b1[...], 0)
    t = jnp.maximum(_dot(t, w2[...]) + b2[...], 0)
    t = jnp.maximum(_dot(t, w3[...]) + b3[...], 0)
    t = _dot(t, w4[...]) + b4[...]
    hn = _ln(t, lg[...], lb[...]) + h[...]
    h_o[...] = hn
    a_o[...] = _dot(hn, wa[...])
    b_o[...] = _dot(hn, wb[...])


def _node_mid_tc(h, a0, a1, wna, wnb, mlp, lnp, wa, wb):
    g = _N // _RB_N
    d = pl.BlockSpec((_RB_N, _LAT), lambda i: (i, 0))
    w = pl.BlockSpec((_LAT, _LAT), lambda i: (0, 0))
    v = pl.BlockSpec((1, _LAT), lambda i: (0, 0))
    return pl.pallas_call(
        _node_mid_body,
        grid=(g,),
        in_specs=[d, d, d, w, w, v, w, v, w, v, w, v, v, v, w, w],
        out_specs=[d, d, d],
        out_shape=[jax.ShapeDtypeStruct((_N, _LAT), _f32)] * 3,
    )(h, a0, a1, wna, wnb, mlp[0]["b"], mlp[1]["W"], mlp[1]["b"],
      mlp[2]["W"], mlp[2]["b"], mlp[3]["W"], mlp[3]["b"],
      lnp["g"], lnp["b"], wa, wb)


def _node_fin_body(h, a0, a1, wna, wnb, b1, w2, b2, w3, b3, w4, b4, lg, lb,
                   d1w, d1b, d2w, d2b, d3w, d3b, d4w, d4b, y_o):
    agg = a0[...] + a1[...]
    t = jnp.maximum(_dot(h[...], wna[...]) + _dot(agg, wnb[...]) + b1[...], 0)
    t = jnp.maximum(_dot(t, w2[...]) + b2[...], 0)
    t = jnp.maximum(_dot(t, w3[...]) + b3[...], 0)
    t = _dot(t, w4[...]) + b4[...]
    hn = _ln(t, lg[...], lb[...]) + h[...]
    t = jnp.maximum(_dot(hn, d1w[...]) + d1b[...], 0)
    t = jnp.maximum(_dot(t, d2w[...]) + d2b[...], 0)
    t = jnp.maximum(_dot(t, d3w[...]) + d3b[...], 0)
    y_o[...] = _dot(t, d4w[...]) + d4b[...]


def _node_fin_tc(h, a0, a1, wna, wnb, mlp, lnp, dec):
    g = _N // _RB_N
    d = pl.BlockSpec((_RB_N, _LAT), lambda i: (i, 0))
    w = pl.BlockSpec((_LAT, _LAT), lambda i: (0, 0))
    v = pl.BlockSpec((1, _LAT), lambda i: (0, 0))
    wo = pl.BlockSpec((_LAT, _OUT), lambda i: (0, 0))
    vo = pl.BlockSpec((1, _OUT), lambda i: (0, 0))
    do = pl.BlockSpec((_RB_N, _OUT), lambda i: (i, 0))
    return pl.pallas_call(
        _node_fin_body,
        grid=(g,),
        in_specs=[d, d, d, w, w, v, w, v, w, v, w, v, v, v,
                  w, v, w, v, w, v, wo, vo],
        out_specs=do,
        out_shape=jax.ShapeDtypeStruct((_N, _OUT), _f32),
    )(h, a0, a1, wna, wnb, mlp[0]["b"], mlp[1]["W"], mlp[1]["b"],
      mlp[2]["W"], mlp[2]["b"], mlp[3]["W"], mlp[3]["b"],
      lnp["g"], lnp["b"],
      dec[0]["W"], dec[0]["b"], dec[1]["W"], dec[1]["b"],
      dec[2]["W"], dec[2]["b"], dec[3]["W"], dec[3]["b"])


def _sc_gather2(a_tab, b_tab, dst, src):
    mesh = plsc.VectorSubcoreMesh(core_axis_name="c", subcore_axis_name="s")
    nch = _EPW // _GC

    @functools.partial(
        pl.kernel, mesh=mesh,
        out_type=jax.ShapeDtypeStruct((_E, _LAT), _f32),
        scratch_types=[pltpu.VMEM((_GC,), jnp.int32),
                       pltpu.VMEM((_GC,), jnp.int32),
                       pltpu.VMEM((_GC,), jnp.int32),
                       pltpu.VMEM((_GC,), jnp.int32),
                       pltpu.VMEM((_GC, _LAT), _f32),
                       pltpu.VMEM((_GC, _LAT), _f32),
                       pltpu.VMEM((_GC, _LAT), _f32),
                       pltpu.VMEM((_GC, _LAT), _f32),
                       pltpu.SemaphoreType.DMA,
                       pltpu.SemaphoreType.DMA,
                       pltpu.SemaphoreType.DMA,
                       pltpu.SemaphoreType.DMA],
    )
    def k(a_hbm, b_hbm, dst_hbm, src_hbm, g_hbm,
          i1a, i2a, i1b, i2b, b1a, b2a, b1b, b2b, s1a, s2a, s1b, s2b):
        wid = lax.axis_index("s") * _NC + lax.axis_index("c")
        idx1 = (i1a, i1b)
        idx2 = (i2a, i2b)
        buf1 = (b1a, b1b)
        buf2 = (b2a, b2b)
        s1 = (s1a, s1b)
        s2 = (s2a, s2b)

        def start(c, sl):
            base = wid * _EPW + c * _GC
            pltpu.sync_copy(dst_hbm.at[pl.ds(base, _GC)], idx1[sl])
            pltpu.sync_copy(src_hbm.at[pl.ds(base, _GC)], idx2[sl])
            pltpu.async_copy(a_hbm.at[idx1[sl]], buf1[sl], s1[sl])
            pltpu.async_copy(b_hbm.at[idx2[sl]], buf2[sl], s2[sl])

        def finish(c, sl):
            base = wid * _EPW + c * _GC
            pltpu.make_async_copy(a_hbm.at[idx1[sl]], buf1[sl], s1[sl]).wait()
            pltpu.make_async_copy(b_hbm.at[idx2[sl]], buf2[sl], s2[sl]).wait()

            def addrow(r, carry):
                for cc in range(_LAT // 16):
                    sl_c = pl.ds(cc * 16, 16)
                    buf1[sl][r, sl_c] = buf1[sl][r, sl_c] + buf2[sl][r, sl_c]
                return carry

            lax.fori_loop(0, _GC, addrow, 0)
            pltpu.sync_copy(buf1[sl], g_hbm.at[pl.ds(base, _GC)])

        start(0, 0)

        def body(j, carry):
            start(2 * j + 1, 1)
            finish(2 * j, 0)
            start(2 * j + 2, 0)
            finish(2 * j + 1, 1)
            return carry

        lax.fori_loop(0, (nch - 1) // 2, body, 0)
        finish(nch - 1, 0)

    return k(a_tab, b_tab, dst, src)


def _sc_segsum(msg, dst):
    mesh = plsc.VectorSubcoreMesh(core_axis_name="c", subcore_axis_name="s")

    @functools.partial(
        pl.kernel, mesh=mesh,
        out_type=[jax.ShapeDtypeStruct((_NP, _LAT), _f32),
                  jax.ShapeDtypeStruct((_NP, _LAT), _f32)],
        scratch_types=[pltpu.VMEM((_GC,), jnp.int32),
                       pltpu.VMEM((_GC, _LAT), _f32),
                       pltpu.VMEM((_ZR, _LAT), _f32),
                       pltpu.VMEM_SHARED((_NP, _LAT), _f32)],
    )
    def k(msg_hbm, dst_hbm, out0_hbm, out1_hbm, idx, buf, zbuf, acc):
        cid = lax.axis_index("c")
        sid = lax.axis_index("s")

        def zb(i, carry):
            zbuf[i // 8, pl.ds((i % 8) * 16, 16)] = jnp.zeros((16,), _f32)
            return carry

        lax.fori_loop(0, _ZR * 8, zb, 0)

        def za(j, carry):
            pltpu.sync_copy(zbuf, acc.at[pl.ds(sid * _RPS + j * _ZR, _ZR)])
            return carry

        lax.fori_loop(0, _RPS // _ZR, za, 0)
        plsc.subcore_barrier()

        base0 = cid * (_E // _NC) + sid * _EPW

        def body(i, carry):
            base = base0 + i * _GC
            pltpu.sync_copy(dst_hbm.at[pl.ds(base, _GC)], idx)
            pltpu.sync_copy(msg_hbm.at[pl.ds(base, _GC)], buf)
            pltpu.sync_copy(buf, acc.at[idx], add=True)
            return carry

        lax.fori_loop(0, _EPW // _GC, body, 0)
        plsc.subcore_barrier()

        @pl.when(cid == 0)
        def _():
            pltpu.sync_copy(acc.at[pl.ds(sid * _RPS, _RPS)],
                            out0_hbm.at[pl.ds(sid * _RPS, _RPS)])

        @pl.when(cid == 1)
        def _():
            pltpu.sync_copy(acc.at[pl.ds(sid * _RPS, _RPS)],
                            out1_hbm.at[pl.ds(sid * _RPS, _RPS)])

    return k(msg, dst)


def kernel(x, edge_attr, params, edge_index):
    src = edge_index[0]
    dst = edge_index[1]
    blocks = params["blocks"]

    def row(v):
        return v.reshape(1, -1)

    def prep_mlp(mlp):
        return [{"W": p["W"], "b": row(p["b"])} for p in mlp]

    def prep_ln(p):
        return {"g": row(p["g"]), "b": row(p["b"])}

    enc_n = prep_mlp(params["enc_node_mlp"])
    enc_e = prep_mlp(params["enc_edge_mlp"])
    dec = prep_mlp(params["dec_mlp"])
    enc_n_ln = prep_ln(params["enc_node_ln"])
    enc_e_ln = prep_ln(params["enc_edge_ln"])

    w1 = [blocks[s]["edge_mlp"][0]["W"] for s in range(2)]
    wa = [w[:_LAT] for w in w1]
    wb = [w[_LAT:2 * _LAT] for w in w1]
    wc = [w[2 * _LAT:] for w in w1]
    wn = [blocks[s]["node_mlp"][0]["W"] for s in range(2)]
    wna = [w[:_LAT] for w in wn]
    wnb = [w[_LAT:] for w in wn]
    e_mlp = [prep_mlp(blocks[s]["edge_mlp"]) for s in range(2)]
    n_mlp = [prep_mlp(blocks[s]["node_mlp"]) for s in range(2)]
    e_ln = [prep_ln(blocks[s]["edge_ln"]) for s in range(2)]
    n_ln = [prep_ln(blocks[s]["node_ln"]) for s in range(2)]

    h, a_tab, b_tab = _enc_node_tc(x, enc_n, enc_n_ln, wa[0], wb[0])
    e = _enc_edge_tc(edge_attr, enc_e, enc_e_ln)

    g1 = _sc_gather2(a_tab, b_tab, dst, src)
    msg, e = _edge_blk_tc(g1, e, wc[0], e_mlp[0], e_ln[0])
    agg0, agg1 = _sc_segsum(msg, dst)
    h, a_tab, b_tab = _node_mid_tc(h, agg0, agg1, wna[0], wnb[0],
                                   n_mlp[0], n_ln[0], wa[1], wb[1])

    g1 = _sc_gather2(a_tab, b_tab, dst, src)
    msg, e = _edge_blk_tc(g1, e, wc[1], e_mlp[1], e_ln[1])
    agg0, agg1 = _sc_segsum(msg, dst)
    y = _node_fin_tc(h, agg0, agg1, wna[1], wnb[1],
                     n_mlp[1], n_ln[1], dec)
    return y


# speedup vs baseline: 4.1508x; 1.1521x over previous
import functools

import jax
import jax.numpy as jnp
from jax import lax
from jax.experimental import pallas as pl
from jax.experimental.pallas import tpu as pltpu
from jax.experimental.pallas import tpu_sc as plsc

_N = 10000
_E = 320000
_LAT = 128
_OUT = 3

_NC = 2
_NS = 16
_NW = _NC * _NS
_EH = _E // 2
_EPW = _EH // _NW
_GC = 40
_NP = 10240
_RPS = _NP // _NS
_ZR = 128

_RB_E = 2000
_RB_N = 2000

_f32 = jnp.float32


def _ln(t, g, b):
    m = jnp.mean(t, axis=-1, keepdims=True)
    v = jnp.mean((t - m) * (t - m), axis=-1, keepdims=True)
    return (t - m) * lax.rsqrt(v + 1e-5) * g + b


def _dot(a, w):
    return jnp.dot(a, w, preferred_element_type=_f32)


def _enc_node_body(x, w1, b1, w2, b2, w3, b3, w4, b4, lg, lb, wa, wb,
                   h_o, a_o, b_o):
    t = jnp.maximum(_dot(x[...], w1[...]) + b1[...], 0)
    t = jnp.maximum(_dot(t, w2[...]) + b2[...], 0)
    t = jnp.maximum(_dot(t, w3[...]) + b3[...], 0)
    t = _dot(t, w4[...]) + b4[...]
    h = _ln(t, lg[...], lb[...])
    h_o[...] = h
    a_o[...] = _dot(h, wa[...])
    b_o[...] = _dot(h, wb[...])


def _enc_node_tc(x, mlp, lnp, wa, wb):
    g = _N // _RB_N
    d = pl.BlockSpec((_RB_N, _LAT), lambda i: (i, 0))
    w = pl.BlockSpec((_LAT, _LAT), lambda i: (0, 0))
    w1 = pl.BlockSpec((_LAT, _LAT), lambda i: (0, 0))
    v = pl.BlockSpec((1, _LAT), lambda i: (0, 0))
    return pl.pallas_call(
        _enc_node_body,
        grid=(g,),
        in_specs=[d, w1, v, w, v, w, v, w, v, v, v, w, w],
        out_specs=[d, d, d],
        out_shape=[jax.ShapeDtypeStruct((_N, _LAT), _f32)] * 3,
    )(x, mlp[0]["W"], mlp[0]["b"], mlp[1]["W"], mlp[1]["b"],
      mlp[2]["W"], mlp[2]["b"], mlp[3]["W"], mlp[3]["b"],
      lnp["g"], lnp["b"], wa, wb)


def _enc_edge_body(ea, w1, b1, w2, b2, w3, b3, w4, b4, lg, lb, e_o):
    t = jnp.maximum(_dot(ea[...], w1[...]) + b1[...], 0)
    t = jnp.maximum(_dot(t, w2[...]) + b2[...], 0)
    t = jnp.maximum(_dot(t, w3[...]) + b3[...], 0)
    t = _dot(t, w4[...]) + b4[...]
    e_o[...] = _ln(t, lg[...], lb[...])


def _enc_edge_tc(ea, mlp, lnp, off):
    g = _EH // _RB_E
    ob = off // _RB_E
    din = pl.BlockSpec((_RB_E, 16), lambda i: (i + ob, 0))
    d = pl.BlockSpec((_RB_E, _LAT), lambda i: (i, 0))
    w16 = pl.BlockSpec((16, _LAT), lambda i: (0, 0))
    w = pl.BlockSpec((_LAT, _LAT), lambda i: (0, 0))
    v = pl.BlockSpec((1, _LAT), lambda i: (0, 0))
    return pl.pallas_call(
        _enc_edge_body,
        grid=(g,),
        in_specs=[din, w16, v, w, v, w, v, w, v, v, v],
        out_specs=d,
        out_shape=jax.ShapeDtypeStruct((_EH, _LAT), _f32),
    )(ea, mlp[0]["W"], mlp[0]["b"], mlp[1]["W"], mlp[1]["b"],
      mlp[2]["W"], mlp[2]["b"], mlp[3]["W"], mlp[3]["b"],
      lnp["g"], lnp["b"])


def _edge_blk_body(g1, e, wc, b1, w2, b2, w3, b3, w4, b4, lg, lb,
                   msg_o, enew_o):
    t = g1[...] + _dot(e[...], wc[...]) + b1[...]
    t = jnp.maximum(t, 0)
    t = jnp.maximum(_dot(t, w2[...]) + b2[...], 0)
    t = jnp.maximum(_dot(t, w3[...]) + b3[...], 0)
    t = _dot(t, w4[...]) + b4[...]
    m = _ln(t, lg[...], lb[...])
    msg_o[...] = m
    enew_o[...] = m + e[...]


def _edge_blk_tc(g1, e, wc, mlp, lnp):
    g = _EH // _RB_E
    d = pl.BlockSpec((_RB_E, _LAT), lambda i: (i, 0))
    w = pl.BlockSpec((_LAT, _LAT), lambda i: (0, 0))
    v = pl.BlockSpec((1, _LAT), lambda i: (0, 0))
    return pl.pallas_call(
        _edge_blk_body,
        grid=(g,),
        in_specs=[d, d, w, v, w, v, w, v, w, v, v, v],
        out_specs=[d, d],
        out_shape=[jax.ShapeDtypeStruct((_EH, _LAT), _f32)] * 2,
    )(g1, e, wc, mlp[0]["b"], mlp[1]["W"], mlp[1]["b"],
      mlp[2]["W"], mlp[2]["b"], mlp[3]["W"], mlp[3]["b"],
      lnp["g"], lnp["b"])


def _node_mid_body(h, a0, a1, a2, a3, wna, wnb, b1, w2, b2, w3, b3, w4, b4,
                   lg, lb, wa, wb, h_o, a_o, b_o):
    agg = (a0[...] + a1[...]) + (a2[...] + a3[...])
    t = jnp.maximum(_dot(h[...], wna[...]) + _dot(agg, wnb[...]) + b1[...], 0)
    t = jnp.maximum(_dot(t, w2[...]) + b2[...], 0)
    t = jnp.maximum(_dot(t, w3[...]) + b3[...], 0)
    t = _dot(t, w4[...]) + b4[...]
    hn = _ln(t, lg[...], lb[...]) + h[...]
    h_o[...] = hn
    a_o[...] = _dot(hn, wa[...])
    b_o[...] = _dot(hn, wb[...])


def _node_mid_tc(h, aggs, wna, wnb, mlp, lnp, wa, wb):
    g = _N // _RB_N
    d = pl.BlockSpec((_RB_N, _LAT), lambda i: (i, 0))
    w = pl.BlockSpec((_LAT, _LAT), lambda i: (0, 0))
    v = pl.BlockSpec((1, _LAT), lambda i: (0, 0))
    return pl.pallas_call(
        _node_mid_body,
        grid=(g,),
        in_specs=[d, d, d, d, d, w, w, v, w, v, w, v, w, v, v, v, w, w],
        out_specs=[d, d, d],
        out_shape=[jax.ShapeDtypeStruct((_N, _LAT), _f32)] * 3,
    )(h, aggs[0], aggs[1], aggs[2], aggs[3], wna, wnb,
      mlp[0]["b"], mlp[1]["W"], mlp[1]["b"],
      mlp[2]["W"], mlp[2]["b"], mlp[3]["W"], mlp[3]["b"],
      lnp["g"], lnp["b"], wa, wb)


def _node_fin_body(h, a0, a1, a2, a3, wna, wnb, b1, w2, b2, w3, b3, w4, b4,
                   lg, lb, d1w, d1b, d2w, d2b, d3w, d3b, d4w, d4b, y_o):
    agg = (a0[...] + a1[...]) + (a2[...] + a3[...])
    t = jnp.maximum(_dot(h[...], wna[...]) + _dot(agg, wnb[...]) + b1[...], 0)
    t = jnp.maximum(_dot(t, w2[...]) + b2[...], 0)
    t = jnp.maximum(_dot(t, w3[...]) + b3[...], 0)
    t = _dot(t, w4[...]) + b4[...]
    hn = _ln(t, lg[...], lb[...]) + h[...]
    t = jnp.maximum(_dot(hn, d1w[...]) + d1b[...], 0)
    t = jnp.maximum(_dot(t, d2w[...]) + d2b[...], 0)
    t = jnp.maximum(_dot(t, d3w[...]) + d3b[...], 0)
    y_o[...] = _dot(t, d4w[...]) + d4b[...]


def _node_fin_tc(h, aggs, wna, wnb, mlp, lnp, dec):
    g = _N // _RB_N
    d = pl.BlockSpec((_RB_N, _LAT), lambda i: (i, 0))
    w = pl.BlockSpec((_LAT, _LAT), lambda i: (0, 0))
    v = pl.BlockSpec((1, _LAT), lambda i: (0, 0))
    wo = pl.BlockSpec((_LAT, _OUT), lambda i: (0, 0))
    vo = pl.BlockSpec((1, _OUT), lambda i: (0, 0))
    do = pl.BlockSpec((_RB_N, _OUT), lambda i: (i, 0))
    return pl.pallas_call(
        _node_fin_body,
        grid=(g,),
        in_specs=[d, d, d, d, d, w, w, v, w, v, w, v, w, v, v, v,
                  w, v, w, v, w, v, wo, vo],
        out_specs=do,
        out_shape=jax.ShapeDtypeStruct((_N, _OUT), _f32),
    )(h, aggs[0], aggs[1], aggs[2], aggs[3], wna, wnb,
      mlp[0]["b"], mlp[1]["W"], mlp[1]["b"],
      mlp[2]["W"], mlp[2]["b"], mlp[3]["W"], mlp[3]["b"],
      lnp["g"], lnp["b"],
      dec[0]["W"], dec[0]["b"], dec[1]["W"], dec[1]["b"],
      dec[2]["W"], dec[2]["b"], dec[3]["W"], dec[3]["b"])


def _sc_gather2(a_tab, b_tab, dst, src, off):
    mesh = plsc.VectorSubcoreMesh(core_axis_name="c", subcore_axis_name="s")
    nch = _EPW // _GC

    @functools.partial(
        pl.kernel, mesh=mesh,
        out_type=jax.ShapeDtypeStruct((_EH, _LAT), _f32),
        scratch_types=[pltpu.VMEM((_GC,), jnp.int32),
                       pltpu.VMEM((_GC,), jnp.int32),
                       pltpu.VMEM((_GC,), jnp.int32),
                       pltpu.VMEM((_GC,), jnp.int32),
                       pltpu.VMEM((_GC, _LAT), _f32),
                       pltpu.VMEM((_GC, _LAT), _f32),
                       pltpu.VMEM((_GC, _LAT), _f32),
                       pltpu.VMEM((_GC, _LAT), _f32),
                       pltpu.SemaphoreType.DMA,
                       pltpu.SemaphoreType.DMA,
                       pltpu.SemaphoreType.DMA,
                       pltpu.SemaphoreType.DMA],
    )
    def k(a_hbm, b_hbm, dst_hbm, src_hbm, g_hbm,
          i1a, i2a, i1b, i2b, b1a, b2a, b1b, b2b, s1a, s2a, s1b, s2b):
        wid = lax.axis_index("s") * _NC + lax.axis_index("c")
        idx1 = (i1a, i1b)
        idx2 = (i2a, i2b)
        buf1 = (b1a, b1b)
        buf2 = (b2a, b2b)
        s1 = (s1a, s1b)
        s2 = (s2a, s2b)

        def start(c, sl):
            base = wid * _EPW + c * _GC
            pltpu.sync_copy(dst_hbm.at[pl.ds(off + base, _GC)], idx1[sl])
            pltpu.sync_copy(src_hbm.at[pl.ds(off + base, _GC)], idx2[sl])
            pltpu.async_copy(a_hbm.at[idx1[sl]], buf1[sl], s1[sl])
            pltpu.async_copy(b_hbm.at[idx2[sl]], buf2[sl], s2[sl])

        def finish(c, sl):
            base = wid * _EPW + c * _GC
            pltpu.make_async_copy(a_hbm.at[idx1[sl]], buf1[sl], s1[sl]).wait()
            pltpu.make_async_copy(b_hbm.at[idx2[sl]], buf2[sl], s2[sl]).wait()

            def addrow(r, carry):
                for cc in range(_LAT // 16):
                    sl_c = pl.ds(cc * 16, 16)
                    buf1[sl][r, sl_c] = buf1[sl][r, sl_c] + buf2[sl][r, sl_c]
                return carry

            lax.fori_loop(0, _GC, addrow, 0)
            pltpu.sync_copy(buf1[sl], g_hbm.at[pl.ds(base, _GC)])

        start(0, 0)

        def body(j, carry):
            start(2 * j + 1, 1)
            finish(2 * j, 0)
            start(2 * j + 2, 0)
            finish(2 * j + 1, 1)
            return carry

        lax.fori_loop(0, (nch - 1) // 2, body, 0)
        finish(nch - 1, 0)

    return k(a_tab, b_tab, dst, src)


def _sc_segsum(msg, dst, off):
    mesh = plsc.VectorSubcoreMesh(core_axis_name="c", subcore_axis_name="s")
    nch = _EPW // _GC

    @functools.partial(
        pl.kernel, mesh=mesh,
        out_type=[jax.ShapeDtypeStruct((_NP, _LAT), _f32),
                  jax.ShapeDtypeStruct((_NP, _LAT), _f32)],
        scratch_types=[pltpu.VMEM((_GC,), jnp.int32),
                       pltpu.VMEM((_GC,), jnp.int32),
                       pltpu.VMEM((_GC, _LAT), _f32),
                       pltpu.VMEM((_GC, _LAT), _f32),
                       pltpu.VMEM((_ZR, _LAT), _f32),
                       pltpu.VMEM_SHARED((_NP, _LAT), _f32),
                       pltpu.SemaphoreType.DMA,
                       pltpu.SemaphoreType.DMA],
    )
    def k(msg_hbm, dst_hbm, out0_hbm, out1_hbm,
          ia, ib, ba, bb, zbuf, acc, sa, sb):
        cid = lax.axis_index("c")
        sid = lax.axis_index("s")
        idx = (ia, ib)
        buf = (ba, bb)
        sem = (sa, sb)

        def zb(i, carry):
            zbuf[i // 8, pl.ds((i % 8) * 16, 16)] = jnp.zeros((16,), _f32)
            return carry

        lax.fori_loop(0, _ZR * 8, zb, 0)

        def za(j, carry):
            pltpu.sync_copy(zbuf, acc.at[pl.ds(sid * _RPS + j * _ZR, _ZR)])
            return carry

        lax.fori_loop(0, _RPS // _ZR, za, 0)
        plsc.subcore_barrier()

        base0 = cid * (_EH // _NC) + sid * _EPW

        def start(c, sl):
            base = base0 + c * _GC
            pltpu.sync_copy(dst_hbm.at[pl.ds(off + base, _GC)], idx[sl])
            pltpu.async_copy(msg_hbm.at[pl.ds(base, _GC)], buf[sl], sem[sl])

        def finish(c, sl):
            base = base0 + c * _GC
            pltpu.make_async_copy(msg_hbm.at[pl.ds(base, _GC)],
                                  buf[sl], sem[sl]).wait()
            pltpu.sync_copy(buf[sl], acc.at[idx[sl]], add=True)

        start(0, 0)

        def body(j, carry):
            start(2 * j + 1, 1)
            finish(2 * j, 0)
            start(2 * j + 2, 0)
            finish(2 * j + 1, 1)
            return carry

        lax.fori_loop(0, (nch - 1) // 2, body, 0)
        finish(nch - 1, 0)
        plsc.subcore_barrier()

        @pl.when(cid == 0)
        def _():
            pltpu.sync_copy(acc.at[pl.ds(sid * _RPS, _RPS)],
                            out0_hbm.at[pl.ds(sid * _RPS, _RPS)])

        @pl.when(cid == 1)
        def _():
            pltpu.sync_copy(acc.at[pl.ds(sid * _RPS, _RPS)],
                            out1_hbm.at[pl.ds(sid * _RPS, _RPS)])

    return k(msg, dst)


def kernel(x, edge_attr, params, edge_index):
    src = edge_index[0]
    dst = edge_index[1]
    blocks = params["blocks"]

    def row(v):
        return v.reshape(1, -1)

    def prep_mlp(mlp):
        return [{"W": p["W"], "b": row(p["b"])} for p in mlp]

    def prep_ln(p):
        return {"g": row(p["g"]), "b": row(p["b"])}

    enc_n = prep_mlp(params["enc_node_mlp"])
    enc_e = prep_mlp(params["enc_edge_mlp"])
    dec = prep_mlp(params["dec_mlp"])
    enc_n_ln = prep_ln(params["enc_node_ln"])
    enc_e_ln = prep_ln(params["enc_edge_ln"])

    w1 = [blocks[s]["edge_mlp"][0]["W"] for s in range(2)]
    wa = [w[:_LAT] for w in w1]
    wb = [w[_LAT:2 * _LAT] for w in w1]
    wc = [w[2 * _LAT:] for w in w1]
    wn = [blocks[s]["node_mlp"][0]["W"] for s in range(2)]
    wna = [w[:_LAT] for w in wn]
    wnb = [w[_LAT:] for w in wn]
    e_mlp = [prep_mlp(blocks[s]["edge_mlp"]) for s in range(2)]
    n_mlp = [prep_mlp(blocks[s]["node_mlp"]) for s in range(2)]
    e_ln = [prep_ln(blocks[s]["edge_ln"]) for s in range(2)]
    n_ln = [prep_ln(blocks[s]["node_ln"]) for s in range(2)]

    h, a_tab, b_tab = _enc_node_tc(x, enc_n, enc_n_ln, wa[0], wb[0])
    e_a = _enc_edge_tc(edge_attr, enc_e, enc_e_ln, 0)
    e_b = _enc_edge_tc(edge_attr, enc_e, enc_e_ln, _EH)

    g_a = _sc_gather2(a_tab, b_tab, dst, src, 0)
    g_b = _sc_gather2(a_tab, b_tab, dst, src, _EH)
    msg_a, e_a = _edge_blk_tc(g_a, e_a, wc[0], e_mlp[0], e_ln[0])
    msg_b, e_b = _edge_blk_tc(g_b, e_b, wc[0], e_mlp[0], e_ln[0])
    p0a, p1a = _sc_segsum(msg_a, dst, 0)
    p0b, p1b = _sc_segsum(msg_b, dst, _EH)
    h, a_tab, b_tab = _node_mid_tc(h, (p0a, p1a, p0b, p1b), wna[0], wnb[0],
                                   n_mlp[0], n_ln[0], wa[1], wb[1])

    g_a = _sc_gather2(a_tab, b_tab, dst, src, 0)
    g_b = _sc_gather2(a_tab, b_tab, dst, src, _EH)
    msg_a, e_a = _edge_blk_tc(g_a, e_a, wc[1], e_mlp[1], e_ln[1])
    msg_b, e_b = _edge_blk_tc(g_b, e_b, wc[1], e_mlp[1], e_ln[1])
    p0a, p1a = _sc_segsum(msg_a, dst, 0)
    p0b, p1b = _sc_segsum(msg_b, dst, _EH)
    y = _node_fin_tc(h, (p0a, p1a, p0b, p1b), wna[1], wnb[1],
                     n_mlp[1], n_ln[1], dec)
    return y
